# Initial kernel scaffold; baseline (speedup 1.0000x reference)
import jax
import jax.numpy as jnp
from jax.experimental import pallas as pl


def kernel(x, Wg_w, Wg_b, comp_w, comp_b):
    raise NotImplementedError("write your pallas kernel here")


# speedup vs baseline: 1.5354x; 1.5354x over previous
import jax
import jax.numpy as jnp
from jax.experimental import pallas as pl


def _mm_kernel(x_ref, w_ref, coef_ref, bias_ref, o_ref):
    acc = jnp.dot(x_ref[...], w_ref[...], preferred_element_type=jnp.float32)
    o_ref[...] = (acc + bias_ref[0]) * coef_ref[0]


def kernel(x, Wg_w, Wg_b, comp_w, comp_b):
    B, D = x.shape
    E, C, _ = comp_w.shape

    logits = x @ Wg_w.T + Wg_b
    p = jax.nn.sigmoid(logits)
    G = jax.random.bernoulli(jax.random.key(42), p).astype(p.dtype)

    active = float(C) * jnp.sum(G, axis=1)
    denom = jnp.where(active > 0, active, 1.0)
    coef = G * (float(C) / denom[:, None])

    xb = x.astype(jnp.bfloat16)
    wb = comp_w.reshape(E * C, D).T.astype(jnp.bfloat16)
    coef3 = coef.T[:, :, None]
    bias3 = comp_b[:, None, :]

    out = pl.pallas_call(
        _mm_kernel,
        grid=(E,),
        in_specs=[
            pl.BlockSpec((B, D), lambda j: (0, 0)),
            pl.BlockSpec((D, C), lambda j: (0, j)),
            pl.BlockSpec((1, B, 1), lambda j: (j, 0, 0)),
            pl.BlockSpec((1, 1, C), lambda j: (j, 0, 0)),
        ],
        out_specs=pl.BlockSpec((B, C), lambda j: (0, j)),
        out_shape=jax.ShapeDtypeStruct((B, E * C), jnp.float32),
    )(xb, wb, coef3, bias3)
    return out, G


# speedup vs baseline: 2.0191x; 1.3150x over previous
import jax
import jax.numpy as jnp
from jax.experimental import pallas as pl


def _mm_kernel(x_ref, w_ref, coef_ref, bias_ref, o_ref):
    w = w_ref[0].astype(jnp.bfloat16)
    acc = jax.lax.dot_general(
        x_ref[...], w, (((1,), (1,)), ((), ())),
        preferred_element_type=jnp.float32)
    o_ref[...] = (acc + bias_ref[0]) * coef_ref[0]


def kernel(x, Wg_w, Wg_b, comp_w, comp_b):
    B, D = x.shape
    E, C, _ = comp_w.shape

    logits = x @ Wg_w.T + Wg_b
    p = jax.nn.sigmoid(logits)
    G = jax.random.bernoulli(jax.random.key(42), p).astype(p.dtype)

    active = float(C) * jnp.sum(G, axis=1)
    denom = jnp.where(active > 0, active, 1.0)
    coef = G * (float(C) / denom[:, None])

    xb = x.astype(jnp.bfloat16)
    coef3 = coef.T[:, :, None]
    bias3 = comp_b[:, None, :]

    out = pl.pallas_call(
        _mm_kernel,
        grid=(E,),
        in_specs=[
            pl.BlockSpec((B, D), lambda j: (0, 0)),
            pl.BlockSpec((1, C, D), lambda j: (j, 0, 0)),
            pl.BlockSpec((1, B, 1), lambda j: (j, 0, 0)),
            pl.BlockSpec((1, 1, C), lambda j: (j, 0, 0)),
        ],
        out_specs=pl.BlockSpec((B, C), lambda j: (0, j)),
        out_shape=jax.ShapeDtypeStruct((B, E * C), jnp.float32),
    )(xb, comp_w, coef3, bias3)
    return out, G


# speedup vs baseline: 2.0581x; 1.0194x over previous
import functools

import jax
import jax.numpy as jnp
from jax.experimental import pallas as pl
from jax.experimental.pallas import tpu as pltpu

_SPLIT = 2


def _fused_kernel(xb_ref, w_ref, wg_ref, thr_ref, bias_ref,
                  o_ref, g_ref, coef_ref, *, split):
    j = pl.program_id(0)
    C = w_ref.shape[1] * split

    @pl.when(j == 0)
    def _gating():
        wg = wg_ref[...].astype(jnp.bfloat16)
        logits = jnp.dot(xb_ref[...], wg,
                         preferred_element_type=jnp.float32)
        g = (logits > thr_ref[...]).astype(jnp.float32)
        g_ref[...] = g
        act = float(C) * jnp.sum(g, axis=1, keepdims=True)
        denom = jnp.where(act > 0.0, act, 1.0)
        coef_ref[...] = g * (float(C) / denom)

    w = w_ref[0].astype(jnp.bfloat16)
    acc = jax.lax.dot_general(
        xb_ref[...], w, (((1,), (1,)), ((), ())),
        preferred_element_type=jnp.float32)
    E = coef_ref.shape[1]
    onehot = (jax.lax.broadcasted_iota(jnp.int32, (1, E), 1) == j // split)
    c = jnp.sum(jnp.where(onehot, coef_ref[...], 0.0),
                axis=1, keepdims=True)
    o_ref[...] = (acc + bias_ref[0]) * c


def kernel(x, Wg_w, Wg_b, comp_w, comp_b):
    B, D = x.shape
    E, C, _ = comp_w.shape
    s = _SPLIT
    BN = C // s

    U = jax.random.uniform(jax.random.key(42), (B, E), jnp.float32)
    thr = jnp.log(U) - jnp.log1p(-U) - Wg_b[None, :]

    xb = x.astype(jnp.bfloat16)
    wg = Wg_w.T
    w4 = comp_w.reshape(E * s, BN, D)
    bias3 = comp_b.reshape(E * s, 1, BN)

    out, G = pl.pallas_call(
        functools.partial(_fused_kernel, split=s),
        grid=(E * s,),
        in_specs=[
            pl.BlockSpec((B, D), lambda j: (0, 0)),
            pl.BlockSpec((1, BN, D), lambda j: (j, 0, 0)),
            pl.BlockSpec((D, E), lambda j: (0, 0)),
            pl.BlockSpec((B, E), lambda j: (0, 0)),
            pl.BlockSpec((1, 1, BN), lambda j: (j, 0, 0)),
        ],
        out_specs=[
            pl.BlockSpec((B, BN), lambda j: (0, j)),
            pl.BlockSpec((B, E), lambda j: (0, 0)),
        ],
        out_shape=[
            jax.ShapeDtypeStruct((B, E * C), jnp.float32),
            jax.ShapeDtypeStruct((B, E), jnp.float32),
        ],
        scratch_shapes=[pltpu.VMEM((B, E), jnp.float32)],
    )(xb, w4, wg, thr, bias3)
    return out, G
